# baseline (speedup 1.0000x reference)
import functools

import jax
import jax.numpy as jnp
from jax import lax
from jax.experimental import pallas as pl
from jax.experimental.pallas import tpu as pltpu
from jax.experimental.pallas import tpu_sc as plsc

_N = 10000
_D = 128
_NNZ = 160000
_NLIVE = 3
_NT = 32
_CH = 128
_EPT = 5120
_EPTOT = _EPT * _NT
_NPAIR = (_EPT // _CH) // 2
_NP = 10240
_RPS = _NP // 16
_DCH = 128


def _mm_body(x_ref, w_ref, o_ref):
    o_ref[:, :] = jnp.dot(x_ref[:, :], w_ref[:, :],
                          preferred_element_type=jnp.float32)


def _matmul(x, w):
    return pl.pallas_call(
        _mm_body,
        grid=(10,),
        in_specs=[pl.BlockSpec((_N // 10, _D), lambda i: (i, 0)),
                  pl.BlockSpec((_D, _D), lambda i: (0, 0))],
        out_specs=pl.BlockSpec((_N // 10, _D), lambda i: (i, 0)),
        out_shape=jax.ShapeDtypeStruct((_N, _D), jnp.float32),
    )(x, w)


def _merge_body(a_ref, o_ref):
    o_ref[:, :] = a_ref[0] + a_ref[1]


def _merge(yp):
    rows = _NLIVE * _NP
    blk = 1024
    return pl.pallas_call(
        _merge_body,
        grid=(rows // blk,),
        in_specs=[pl.BlockSpec((2, blk, _D), lambda i: (0, i, 0))],
        out_specs=pl.BlockSpec((blk, _D), lambda i: (i, 0)),
        out_shape=jax.ShapeDtypeStruct((rows, _D), jnp.float32),
    )(yp)


def _final_body(a_ref, b_ref, o_ref):
    o_ref[:, :] = a_ref[0] + a_ref[1] + b_ref[:, :]


def _final(op, bias2d):
    blk = 1000
    return pl.pallas_call(
        _final_body,
        grid=(_N // blk,),
        in_specs=[pl.BlockSpec((2, blk, _D), lambda i: (0, i, 0)),
                  pl.BlockSpec((1, _D), lambda i: (0, 0))],
        out_specs=pl.BlockSpec((blk, _D), lambda i: (i, 0)),
        out_shape=jax.ShapeDtypeStruct((_N, _D), jnp.float32),
    )(op, bias2d)


_MESH = dict(core_axis_name="c", subcore_axis_name="s")
_SC_PARAMS = dict(
    compiler_params=pltpu.CompilerParams(needs_layout_passes=False))


def _zero_buf(buf):
    @plsc.parallel_loop(0, _DCH)
    def _(i):
        for j in range(_D // 16):
            buf[i, pl.ds(j * 16, 16)] = jnp.zeros((16,), jnp.float32)


def _scale_rows(gbuf, vals_v):
    @plsc.parallel_loop(0, _CH, unroll=4)
    def _(e):
        e16 = jnp.full((16,), 0, jnp.int32) + e
        s16 = plsc.load_gather(vals_v, [e16])
        for c in range(_D // 16):
            gbuf[e, pl.ds(c * 16, 16)] = gbuf[e, pl.ds(c * 16, 16)] * s16


_IDXBUFS = lambda: [pltpu.VMEM((_CH,), jnp.int32),
                    pltpu.VMEM((_CH,), jnp.int32),
                    pltpu.VMEM((_CH,), jnp.float32)]


def _sc_stage1(h, rows_p, cols_p, vals_p, filt_flat):
    mesh = plsc.VectorSubcoreMesh(**_MESH)

    @functools.partial(
        pl.kernel,
        out_type=jax.ShapeDtypeStruct((2, _NLIVE, _NP, _D), jnp.float32),
        mesh=mesh,
        scratch_types=[
            pltpu.VMEM_SHARED((_NP, _D), jnp.float32),
            *_IDXBUFS(), *_IDXBUFS(),
            pltpu.VMEM((_CH, _D), jnp.float32),
            pltpu.VMEM((_CH, _D), jnp.float32),
            pltpu.VMEM((_N,), jnp.float32),
            pltpu.SemaphoreType.DMA,
            pltpu.SemaphoreType.DMA,
        ],
        **_SC_PARAMS,
    )
    def k(h_hbm, erows_hbm, ecols_hbm, evals_hbm, filt_hbm, yp_hbm,
          ysp, rows0, cols0, vals0, rows1, cols1, vals1,
          gbuf0, gbuf1, filt_v, sem0, sem1):
        cid = lax.axis_index("c")
        sid = lax.axis_index("s")
        tid = cid * 16 + sid
        base = tid * _EPT

        _zero_buf(gbuf0)
        for i in range(_RPS // _DCH):
            pltpu.sync_copy(gbuf0, ysp.at[pl.ds(sid * _RPS + i * _DCH, _DCH)])
        plsc.subcore_barrier()

        for mm in range(_NLIVE):
            pltpu.sync_copy(filt_hbm.at[pl.ds((mm + 1) * _N, _N)], filt_v)

            def load_idx(g, rows_v, cols_v, vals_v):
                off = mm * _EPTOT + base + g * _CH
                pltpu.sync_copy(erows_hbm.at[pl.ds(off, _CH)], rows_v)
                pltpu.sync_copy(ecols_hbm.at[pl.ds(off, _CH)], cols_v)
                pltpu.sync_copy(evals_hbm.at[pl.ds(off, _CH)], vals_v)

            def fold_filt(rows_v, vals_v):
                @plsc.parallel_loop(0, _CH, step=16)
                def _(j):
                    r16 = rows_v[pl.ds(j, 16)]
                    f16 = plsc.load_gather(filt_v, [r16])
                    vals_v[pl.ds(j, 16)] = vals_v[pl.ds(j, 16)] * f16

            def finish(rows_v, vals_v, gbuf):
                fold_filt(rows_v, vals_v)
                _scale_rows(gbuf, vals_v)
                pltpu.sync_copy(gbuf, ysp.at[rows_v], add=True)

            load_idx(0, rows0, cols0, vals0)
            pltpu.async_copy(h_hbm.at[cols0], gbuf0, sem0)

            def pair(p, carry):
                load_idx(2 * p + 1, rows1, cols1, vals1)
                pltpu.async_copy(h_hbm.at[cols1], gbuf1, sem1)
                pltpu.make_async_copy(h_hbm.at[cols0], gbuf0, sem0).wait()
                finish(rows0, vals0, gbuf0)

                @pl.when(p < _NPAIR - 1)
                def _():
                    load_idx(2 * p + 2, rows0, cols0, vals0)
                    pltpu.async_copy(h_hbm.at[cols0], gbuf0, sem0)

                pltpu.make_async_copy(h_hbm.at[cols1], gbuf1, sem1).wait()
                finish(rows1, vals1, gbuf1)
                return carry

            lax.fori_loop(0, _NPAIR, pair, 0)

            plsc.subcore_barrier()
            _zero_buf(gbuf0)
            for i in range(_RPS // _DCH):
                start = sid * _RPS + i * _DCH
                pltpu.sync_copy(ysp.at[pl.ds(start, _DCH)],
                                yp_hbm.at[cid, mm, pl.ds(start, _DCH)])
                pltpu.sync_copy(gbuf0, ysp.at[pl.ds(start, _DCH)])
            plsc.subcore_barrier()

    return k(h, rows_p, cols_p, vals_p, filt_flat)


def _sc_stage2(ym, rows_p, cols_p, vals_p):
    mesh = plsc.VectorSubcoreMesh(**_MESH)

    @functools.partial(
        pl.kernel,
        out_type=jax.ShapeDtypeStruct((2, _NP, _D), jnp.float32),
        mesh=mesh,
        scratch_types=[
            pltpu.VMEM_SHARED((_NP, _D), jnp.float32),
            *_IDXBUFS(), *_IDXBUFS(),
            pltpu.VMEM((_CH, _D), jnp.float32),
            pltpu.VMEM((_CH, _D), jnp.float32),
            pltpu.SemaphoreType.DMA,
            pltpu.SemaphoreType.DMA,
        ],
        **_SC_PARAMS,
    )
    def k(ym_hbm, erows_hbm, ecols_hbm, evals_hbm, op_hbm,
          osp, rows0, cols0, vals0, rows1, cols1, vals1,
          gbuf0, gbuf1, sem0, sem1):
        cid = lax.axis_index("c")
        sid = lax.axis_index("s")
        tid = cid * 16 + sid
        base = tid * _EPT

        _zero_buf(gbuf0)
        for i in range(_RPS // _DCH):
            pltpu.sync_copy(gbuf0, osp.at[pl.ds(sid * _RPS + i * _DCH, _DCH)])
        plsc.subcore_barrier()

        for mm in range(_NLIVE):
            yoff = mm * _NP

            def load_idx(g, rows_v, cols_v, vals_v):
                off = mm * _EPTOT + base + g * _CH
                pltpu.sync_copy(erows_hbm.at[pl.ds(off, _CH)], rows_v)
                pltpu.sync_copy(ecols_hbm.at[pl.ds(off, _CH)], cols_v)
                pltpu.sync_copy(evals_hbm.at[pl.ds(off, _CH)], vals_v)

                @plsc.parallel_loop(0, _CH, step=16)
                def _(j):
                    cols_v[pl.ds(j, 16)] = cols_v[pl.ds(j, 16)] + yoff

            def finish(rows_v, vals_v, gbuf):
                _scale_rows(gbuf, vals_v)
                pltpu.sync_copy(gbuf, osp.at[rows_v], add=True)

            load_idx(0, rows0, cols0, vals0)
            pltpu.async_copy(ym_hbm.at[cols0], gbuf0, sem0)

            def pair(p, carry):
                load_idx(2 * p + 1, rows1, cols1, vals1)
                pltpu.async_copy(ym_hbm.at[cols1], gbuf1, sem1)
                pltpu.make_async_copy(ym_hbm.at[cols0], gbuf0, sem0).wait()
                finish(rows0, vals0, gbuf0)

                @pl.when(p < _NPAIR - 1)
                def _():
                    load_idx(2 * p + 2, rows0, cols0, vals0)
                    pltpu.async_copy(ym_hbm.at[cols0], gbuf0, sem0)

                pltpu.make_async_copy(ym_hbm.at[cols1], gbuf1, sem1).wait()
                finish(rows1, vals1, gbuf1)
                return carry

            lax.fori_loop(0, _NPAIR, pair, 0)

        plsc.subcore_barrier()
        for i in range(_RPS // _DCH):
            start = sid * _RPS + i * _DCH
            pltpu.sync_copy(osp.at[pl.ds(start, _DCH)],
                            op_hbm.at[cid, pl.ds(start, _DCH)])

    return k(ym, rows_p, cols_p, vals_p)


def kernel(x, d_indices, d_values, weight, filt, bias):
    pad = ((0, 0), (0, _EPTOT - _NNZ))
    rows_p = jnp.pad(d_indices[1:, 0, :], pad).reshape(-1)
    cols_p = jnp.pad(d_indices[1:, 1, :], pad).reshape(-1)
    vals_p = jnp.pad(d_values[1:], pad).reshape(-1)
    h = _matmul(x, weight)
    yp = _sc_stage1(h, rows_p, cols_p, vals_p, filt.reshape(-1))
    ym = _merge(yp.reshape(2, _NLIVE * _NP, _D))
    op = _sc_stage2(ym, rows_p, cols_p, vals_p)
    return _final(op, bias.reshape(1, _D))
